# Initial kernel scaffold; baseline (speedup 1.0000x reference)
import jax
import jax.numpy as jnp
from jax.experimental import pallas as pl


def kernel(features, conv_w, conv_b, cls_w, cls_b, bbox_w, bbox_b):
    raise NotImplementedError("write your pallas kernel here")


# speedup vs baseline: 7.9602x; 7.9602x over previous
import math

import jax
import jax.numpy as jnp
import numpy as np
from jax.experimental import pallas as pl
from jax.experimental.pallas import tpu as pltpu

B, C, H, W = 4, 256, 50, 50
A = 15
STRIDE = 16
IMG = 800.0
PRE_NMS = 2000
POST_NMS = 1000
NMS_THRESH = np.float32(0.7)
BBOX_CLAMP = np.float32(math.log(1000.0 / 16.0))
MIN_SIZE = np.float32(1e-3)

HW = H * W
HWP = 2560
NT = 5
TW = HWP // NT
KP = 2048

_scales = np.array([32., 64., 128., 256., 512.], np.float32)
_ratios = np.array([0.5, 1.0, 2.0], np.float32)
_h_r = np.sqrt(_ratios).astype(np.float32)
_w_r = (np.float32(1.0) / _h_r).astype(np.float32)
_ws = (_w_r[:, None] * _scales[None, :]).reshape(-1)
_hs = (_h_r[:, None] * _scales[None, :]).reshape(-1)
_cell = np.round(np.stack([-_ws, -_hs, _ws, _hs], axis=1).astype(np.float32) / 2.0)
_AW = (_cell[:, 2] - _cell[:, 0]).astype(np.float32)
_AH = (_cell[:, 3] - _cell[:, 1]).astype(np.float32)
_ACX = (_cell[:, 0] + 0.5 * _AW).astype(np.float32)
_ACY = (_cell[:, 1] + 0.5 * _AH).astype(np.float32)


_ANCH = np.zeros((16, 4), np.float32)
_ANCH[:A, 0] = _AW
_ANCH[:A, 1] = _AH
_ANCH[:A, 2] = _ACX
_ANCH[:A, 3] = _ACY


def _head_kernel(anch_ref, xcol_ref, wmat_ref, convb_ref, clsw_ref, clsb_ref,
                 bboxw_ref, bboxb_ref, scores_ref, boxes_ref):
    t_idx = pl.program_id(1)
    x = xcol_ref[0]
    t = jnp.dot(wmat_ref[:, :], x, preferred_element_type=jnp.float32)
    t = jnp.maximum(t + convb_ref[:, :], 0.0)
    logits = jnp.dot(clsw_ref[:, :], t, preferred_element_type=jnp.float32)
    logits = logits + clsb_ref[:, :]
    d = jnp.dot(bboxw_ref[:, :], t, preferred_element_type=jnp.float32)
    d = d + bboxb_ref[:, :]
    dx, dy = d[0:A], d[A:2 * A]
    dw = jnp.minimum(d[2 * A:3 * A], BBOX_CLAMP)
    dh = jnp.minimum(d[3 * A:4 * A], BBOX_CLAMP)

    hw = t_idx * TW + jax.lax.broadcasted_iota(jnp.int32, (1, TW), 1)
    col = (hw % W).astype(jnp.float32) * np.float32(STRIDE)
    row = (hw // W).astype(jnp.float32) * np.float32(STRIDE)

    aw = anch_ref[0:A, 0:1]
    ah = anch_ref[0:A, 1:2]
    cx = col + anch_ref[0:A, 2:3]
    cy = row + anch_ref[0:A, 3:4]
    pcx = dx * aw + cx
    pcy = dy * ah + cy
    pw = jnp.exp(dw) * aw
    ph = jnp.exp(dh) * ah
    x0 = jnp.clip(pcx - 0.5 * pw, 0.0, IMG)
    y0 = jnp.clip(pcy - 0.5 * ph, 0.0, IMG)
    x1 = jnp.clip(pcx + 0.5 * pw, 0.0, IMG)
    y1 = jnp.clip(pcy + 0.5 * ph, 0.0, IMG)
    boxes_ref[0, 0:A] = x0
    boxes_ref[0, A:2 * A] = y0
    boxes_ref[0, 2 * A:3 * A] = x1
    boxes_ref[0, 3 * A:4 * A] = y1

    sig = jax.nn.sigmoid(logits[0:A])
    valid = ((x1 - x0) >= MIN_SIZE) & ((y1 - y0) >= MIN_SIZE)
    scores_ref[0, 0:A] = jnp.where(valid, sig, -1.0)
    scores_ref[0, A:A + 1] = jnp.full((1, TW), -1.0, jnp.float32)


def _nms_kernel(bv_ref, bs_ref, sc_ref, out_ref):
    x0 = bv_ref[0, 0]
    y0 = bv_ref[0, 1]
    x1 = bv_ref[0, 2]
    y1 = bv_ref[0, 3]
    area = (x1 - x0) * (y1 - y0)
    idxv = (jax.lax.broadcasted_iota(jnp.int32, (16, 128), 0) * 128
            + jax.lax.broadcasted_iota(jnp.int32, (16, 128), 1))

    def body(i, keep):
        x0i = bs_ref[0, i, 0]
        y0i = bs_ref[0, i, 1]
        x1i = bs_ref[0, i, 2]
        y1i = bs_ref[0, i, 3]
        ai = (x1i - x0i) * (y1i - y0i)
        keep_i = jnp.max(jnp.where(idxv == i, keep, 0.0))
        iw = jnp.maximum(jnp.minimum(x1, x1i) - jnp.maximum(x0, x0i), 0.0)
        ih = jnp.maximum(jnp.minimum(y1, y1i) - jnp.maximum(y0, y0i), 0.0)
        inter = iw * ih
        union = ((ai + area) - inter) + np.float32(1e-9)
        supp = (inter > NMS_THRESH * union) & (idxv > i)
        return jnp.where(supp & (keep_i > 0.0), 0.0, keep)

    keep = jax.lax.fori_loop(0, PRE_NMS, body, jnp.ones((16, 128), jnp.float32))
    out_ref[0] = jnp.where((keep > 0.0) & (idxv < PRE_NMS), sc_ref[0], -1.0)


def kernel(features, conv_w, conv_b, cls_w, cls_b, bbox_w, bbox_b):
    f32 = jnp.float32
    xpad = jnp.pad(features, ((0, 0), (0, 0), (1, 1), (1, 1)))
    cols = [xpad[:, :, dy:dy + H, dx:dx + W]
            for dy in range(3) for dx in range(3)]
    xcol = jnp.stack(cols, axis=1).reshape(B, 9 * C, HW)
    xcol = jnp.pad(xcol, ((0, 0), (0, 0), (0, HWP - HW)))
    wmat = conv_w.transpose(0, 2, 3, 1).reshape(C, 9 * C)
    convb = conv_b.reshape(C, 1)
    clsw = jnp.pad(cls_w.reshape(A, C), ((0, 1), (0, 0)))
    clsb = jnp.pad(cls_b, (0, 1)).reshape(A + 1, 1)
    bw = bbox_w.reshape(A, 4, C).transpose(1, 0, 2).reshape(4 * A, C)
    bboxw = jnp.pad(bw, ((0, 4), (0, 0)))
    bb = bbox_b.reshape(A, 4).transpose(1, 0).reshape(4 * A)
    bboxb = jnp.pad(bb, (0, 4)).reshape(4 * A + 4, 1)

    scores_g, boxes_g = pl.pallas_call(
        _head_kernel,
        grid=(B, NT),
        in_specs=[
            pl.BlockSpec((16, 4), lambda b, t: (0, 0)),
            pl.BlockSpec((1, 9 * C, TW), lambda b, t: (b, 0, t)),
            pl.BlockSpec((C, 9 * C), lambda b, t: (0, 0)),
            pl.BlockSpec((C, 1), lambda b, t: (0, 0)),
            pl.BlockSpec((A + 1, C), lambda b, t: (0, 0)),
            pl.BlockSpec((A + 1, 1), lambda b, t: (0, 0)),
            pl.BlockSpec((4 * A + 4, C), lambda b, t: (0, 0)),
            pl.BlockSpec((4 * A + 4, 1), lambda b, t: (0, 0)),
        ],
        out_specs=[
            pl.BlockSpec((1, A + 1, TW), lambda b, t: (b, 0, t)),
            pl.BlockSpec((1, 4 * A + 4, TW), lambda b, t: (b, 0, t)),
        ],
        out_shape=[
            jax.ShapeDtypeStruct((B, A + 1, HWP), f32),
            jax.ShapeDtypeStruct((B, 4 * A + 4, HWP), f32),
        ],
        compiler_params=pltpu.CompilerParams(
            dimension_semantics=("parallel", "parallel")),
    )(jnp.asarray(_ANCH), xcol, wmat, convb, clsw, clsb, bboxw, bboxb)

    scores = scores_g[:, :A, :HW].transpose(0, 2, 1).reshape(B, HW * A)
    boxes = (boxes_g[:, :4 * A, :HW].reshape(B, 4, A, HW)
             .transpose(0, 3, 2, 1).reshape(B, HW * A, 4))

    top_s, top_i = jax.lax.top_k(scores, PRE_NMS)
    boxes_top = jnp.take_along_axis(boxes, top_i[:, :, None], axis=1)
    bt = jnp.pad(boxes_top, ((0, 0), (0, KP - PRE_NMS), (0, 0)))
    bv = bt.transpose(0, 2, 1).reshape(B, 4, 16, 128)
    ts = jnp.pad(top_s, ((0, 0), (0, KP - PRE_NMS)),
                 constant_values=-1.0).reshape(B, 16, 128)

    kept = pl.pallas_call(
        _nms_kernel,
        grid=(B,),
        in_specs=[
            pl.BlockSpec((1, 4, 16, 128), lambda b: (b, 0, 0, 0)),
            pl.BlockSpec((1, KP, 4), lambda b: (b, 0, 0)),
            pl.BlockSpec((1, 16, 128), lambda b: (b, 0, 0)),
        ],
        out_specs=pl.BlockSpec((1, 16, 128), lambda b: (b, 0, 0)),
        out_shape=jax.ShapeDtypeStruct((B, 16, 128), f32),
        compiler_params=pltpu.CompilerParams(
            dimension_semantics=("parallel",)),
    )(bv, bt, ts)

    kept_scores = kept.reshape(B, KP)
    _, sel = jax.lax.top_k(kept_scores, POST_NMS)
    return jnp.take_along_axis(bt, sel[:, :, None], axis=1)
